# Initial kernel scaffold; baseline (speedup 1.0000x reference)
import jax
import jax.numpy as jnp
from jax.experimental import pallas as pl


def kernel(workclass, education, marital_status, occupation, relationship, race, sex, native_country, emb_mean_workclass, emb_std_workclass, emb_mean_education, emb_std_education, emb_mean_marital_status, emb_std_marital_status, emb_mean_occupation, emb_std_occupation, emb_mean_relationship, emb_std_relationship, emb_mean_race, emb_std_race, emb_mean_sex, emb_std_sex, emb_mean_native_country, emb_std_native_country, fc_00, fc_01, fc_02, fc_03, fc_04, fc_05, fc_06, fc_07, fc_08, fc_09, fc_10, fc_11, fc_12, fc_13, fc_14, fc_15, fc_16, fc_17, fc_18, fc_19, fc_20, fc_21, fc_22, fc_23, fc_24, fc_25, fc_26, fc_27, p_w1, p_b1, p_w2, p_b2, q_w1, q_b1, q_w2, q_b2, v):
    raise NotImplementedError("write your pallas kernel here")


# speedup vs baseline: 3.0185x; 3.0185x over previous
import functools

import jax
import jax.numpy as jnp
from jax.experimental import pallas as pl
from jax.experimental.pallas import tpu as pltpu

_COLS = 8
_EMB = 128
_ROWS = 12
_RPAD = 16
_B = 4096

_POS = [3, 0, 3, 2, 2, 2, 3, 4, 4, 3, 0, 0, 0, 3,
        0, 3, 0, 3, 4, 4, 2, 4, 3, 4, 3, 2, 2, 3]
_PAIRS = [(i1, i2) for i1 in range(_COLS) for i2 in range(_COLS) if i1 < i2]
_DIFF = [(i1, i2, (0.5 if p == 2 else -0.5))
         for (i1, i2), p in zip(_PAIRS, _POS) if p in (2, 3)]

_NP = 7
_NQ = 7
_KU = (_NP + _NQ + len(_DIFF)) * _EMB


def _fused_kernel(idx_ref, tmu_ref, tstd_ref, v_ref, pw_ref, qw_ref,
                  wcat_ref, out_ref, *, blk):
    f32 = jnp.float32
    idx8 = idx_ref[...]
    idxe = jnp.tile(idx8, (1, _RPAD))
    lane = jax.lax.broadcasted_iota(jnp.int32, (blk, _RPAD * _COLS), 1)
    onehot = (idxe == (lane // _COLS)).astype(f32)

    tsp = jnp.log(1.0 + jnp.exp(tstd_ref[...]))
    g_mu = jnp.dot(onehot, tmu_ref[...], preferred_element_type=f32)
    g_sp = jnp.dot(onehot, tsp, preferred_element_type=f32)

    vt = jnp.tile(v_ref[...], (1, _COLS))
    e_all = g_mu + g_sp * vt * 0.01

    e_p = e_all[:, : _NP * _EMB]
    e_q = e_all[:, _EMB:]
    p_acc = jnp.zeros_like(e_p)
    q_acc = jnp.zeros_like(e_q)
    for j in range(8):
        p_acc += pw_ref[2, j] * jnp.tanh(e_p * pw_ref[0, j] + pw_ref[1, j])
        q_acc += qw_ref[2, j] * jnp.tanh(e_q * qw_ref[0, j] + qw_ref[1, j])
    p_out = p_acc + pw_ref[3, 0]
    q_out = q_acc + qw_ref[3, 0]

    diffs = [jnp.abs(p_out[:, i1 * _EMB:(i1 + 1) * _EMB]
                     - q_out[:, (i2 - 1) * _EMB:i2 * _EMB])
             for (i1, i2, _s) in _DIFF]

    u = jnp.concatenate([p_out, q_out] + diffs, axis=1)
    out_ref[...] = jnp.dot(u, wcat_ref[...], preferred_element_type=f32)


def kernel(workclass, education, marital_status, occupation, relationship,
           race, sex, native_country,
           emb_mean_workclass, emb_std_workclass, emb_mean_education,
           emb_std_education, emb_mean_marital_status, emb_std_marital_status,
           emb_mean_occupation, emb_std_occupation, emb_mean_relationship,
           emb_std_relationship, emb_mean_race, emb_std_race,
           emb_mean_sex, emb_std_sex, emb_mean_native_country,
           emb_std_native_country,
           fc_00, fc_01, fc_02, fc_03, fc_04, fc_05, fc_06, fc_07, fc_08,
           fc_09, fc_10, fc_11, fc_12, fc_13, fc_14, fc_15, fc_16, fc_17,
           fc_18, fc_19, fc_20, fc_21, fc_22, fc_23, fc_24, fc_25, fc_26,
           fc_27,
           p_w1, p_b1, p_w2, p_b2, q_w1, q_b1, q_w2, q_b2, v):
    f32 = jnp.float32
    idx_list = [workclass, education, marital_status, occupation,
                relationship, race, sex, native_country]
    means = [emb_mean_workclass, emb_mean_education, emb_mean_marital_status,
             emb_mean_occupation, emb_mean_relationship, emb_mean_race,
             emb_mean_sex, emb_mean_native_country]
    stds = [emb_std_workclass, emb_std_education, emb_std_marital_status,
            emb_std_occupation, emb_std_relationship, emb_std_race,
            emb_std_sex, emb_std_native_country]
    fcs = [fc_00, fc_01, fc_02, fc_03, fc_04, fc_05, fc_06, fc_07, fc_08,
           fc_09, fc_10, fc_11, fc_12, fc_13, fc_14, fc_15, fc_16, fc_17,
           fc_18, fc_19, fc_20, fc_21, fc_22, fc_23, fc_24, fc_25, fc_26,
           fc_27]

    idx8 = jnp.stack(idx_list, axis=1).astype(jnp.int32)

    def layout(tabs, fill):
        t = jnp.stack([jnp.pad(x, ((0, _RPAD - _ROWS), (0, 0)),
                               constant_values=fill) for x in tabs])
        rows = jnp.transpose(t, (1, 0, 2)).reshape(_RPAD * _COLS, _EMB)
        mask = (jnp.arange(_RPAD * _COLS)[:, None] % _COLS
                == jnp.arange(_COLS)[None, :])
        full = jnp.where(mask[:, :, None], rows[:, None, :], fill)
        return full.reshape(_RPAD * _COLS, _COLS * _EMB).astype(f32)

    t_mu = layout(means, 0.0)
    t_std = layout(stds, -1e30)

    wp = [jnp.zeros((_EMB, 2), f32) for _ in range(_COLS)]
    wq = [jnp.zeros((_EMB, 2), f32) for _ in range(_COLS)]
    wd = []
    for k, (i1, i2) in enumerate(_PAIRS):
        pos = _POS[k]
        fc = fcs[k].astype(f32)
        if pos == 0:
            wp[i1] += fc.T
            wq[i2] += fc.T
        elif pos == 1:
            raise NotImplementedError
        elif pos in (2, 3):
            wp[i1] += 0.5 * fc.T
            wq[i2] += 0.5 * fc.T
        else:
            wp[i1] += fc[:, :_EMB].T
            wq[i2] += fc[:, _EMB:].T
    for (i1, i2), pos in zip(_PAIRS, _POS):
        if pos in (2, 3):
            wd.append((0.5 if pos == 2 else -0.5) * fcs[_PAIRS.index((i1, i2))].T)
    w_cat = jnp.concatenate(wp[:_NP] + wq[1:] + wd, axis=0)

    pw = jnp.stack([p_w1.reshape(8), p_b1.reshape(8), p_w2.reshape(8),
                    jnp.tile(p_b2.reshape(1), 8)]).astype(f32)
    qw = jnp.stack([q_w1.reshape(8), q_b1.reshape(8), q_w2.reshape(8),
                    jnp.tile(q_b2.reshape(1), 8)]).astype(f32)

    blk = 512
    grid = (_B // blk,)
    out = pl.pallas_call(
        functools.partial(_fused_kernel, blk=blk),
        grid=grid,
        in_specs=[
            pl.BlockSpec((blk, _COLS), lambda i: (i, 0)),
            pl.BlockSpec((_RPAD * _COLS, _COLS * _EMB), lambda i: (0, 0)),
            pl.BlockSpec((_RPAD * _COLS, _COLS * _EMB), lambda i: (0, 0)),
            pl.BlockSpec((blk, _EMB), lambda i: (i, 0)),
            pl.BlockSpec(memory_space=pltpu.SMEM),
            pl.BlockSpec(memory_space=pltpu.SMEM),
            pl.BlockSpec((_KU, 2), lambda i: (0, 0)),
        ],
        out_specs=pl.BlockSpec((blk, 2), lambda i: (i, 0)),
        out_shape=jax.ShapeDtypeStruct((_B, 2), f32),
        compiler_params=pltpu.CompilerParams(
            dimension_semantics=("arbitrary",),
        ),
    )(idx8, t_mu, t_std, v.astype(f32), pw, qw, w_cat)
    return out
